# baseline (speedup 1.0000x reference)
import functools

import numpy as np
import jax
import jax.numpy as jnp
from jax import lax
from jax.experimental import pallas as pl
from jax.experimental.pallas import tpu as pltpu
from jax.experimental.pallas import tpu_sc as plsc

NUM_LEVEL = 16
T = 2 ** 19
FEAT_DIM = 2
N_PTS = 262144
GEO_DIM = 16
HIDDEN = 64

NC, NS = 2, 16
NW = NC * NS
PTS_PER_TILE = N_PTS // NW
CHUNK = 1024
NCHUNK = PTS_PER_TILE // CHUNK
TMASK = T - 1
HC1 = int(np.int32(np.uint32(2654435761)))
HC2 = int(np.int32(np.uint32(805459861)))
_RES = [float(r) for r in np.floor(16.0 * (128.0 ** (1.0 / 15.0)) ** np.arange(16))]


def _sc_embed(coords, tables_words):
    mesh = plsc.VectorSubcoreMesh(core_axis_name="c", subcore_axis_name="s")

    @functools.partial(
        pl.kernel,
        out_type=jax.ShapeDtypeStruct((2 * NUM_LEVEL * N_PTS,), jnp.float32),
        mesh=mesh,
        scratch_types=[
            pltpu.VMEM((CHUNK, 3), jnp.float32),
            pltpu.VMEM((2, 6, CHUNK), jnp.float32),
            pltpu.VMEM((2, 16, CHUNK), jnp.int32),
            pltpu.VMEM((2, 16, CHUNK), jnp.float32),
            pltpu.VMEM((2 * NUM_LEVEL, CHUNK), jnp.float32),
            pltpu.SemaphoreType.DMA,
            pltpu.SemaphoreType.DMA,
            pltpu.SemaphoreType.DMA,
        ],
        compiler_params=pltpu.CompilerParams(
            use_tc_tiling_on_sc=False, needs_layout_passes=False),
    )
    def k(coords_hbm, tables_hbm, feats_hbm, cbuf, wbuf, ibuf, gbuf, facc,
          gsem0, gsem1, osem):
        wid = lax.axis_index("s") * NC + lax.axis_index("c")
        gsems = (gsem0, gsem1)
        iota16 = lax.iota(jnp.int32, 16)
        cd0 = jnp.zeros((16,), jnp.int32)
        cd1 = jnp.full((16,), 1, jnp.int32)
        cd2 = jnp.full((16,), 2, jnp.int32)

        def hash_fire(l, buf):
            res = _RES[l]
            off2 = 2 * l * T

            def hi(i, _):
                ds = pl.ds(i * 16, 16)
                pidx = iota16 + i * 16
                sx = plsc.load_gather(cbuf, [pidx, cd0]) * res
                sy = plsc.load_gather(cbuf, [pidx, cd1]) * res
                sz = plsc.load_gather(cbuf, [pidx, cd2]) * res
                fx = sx.astype(jnp.int32)
                fy = sy.astype(jnp.int32)
                fz = sz.astype(jnp.int32)
                frx = sx - fx.astype(jnp.float32)
                fry = sy - fy.astype(jnp.float32)
                frz = sz - fz.astype(jnp.float32)
                wbuf[buf, 0, ds] = 1.0 - frx
                wbuf[buf, 1, ds] = frx
                wbuf[buf, 2, ds] = 1.0 - fry
                wbuf[buf, 3, ds] = fry
                wbuf[buf, 4, ds] = 1.0 - frz
                wbuf[buf, 5, ds] = frz
                hy0 = fy * HC1
                hy1 = hy0 + HC1
                hz0 = fz * HC2
                hz1 = hz0 + HC2
                hx1 = fx + 1
                a = (fx ^ hy0, hx1 ^ hy0, fx ^ hy1, hx1 ^ hy1)
                for m in range(8):
                    hxy = a[m & 3]
                    hz = hz0 if (m >> 2) & 1 == 0 else hz1
                    w0 = (((hxy ^ hz) & TMASK) << 1) + off2
                    ibuf[buf, 2 * m, ds] = w0
                    ibuf[buf, 2 * m + 1, ds] = w0 + 1
                return 0

            lax.fori_loop(0, CHUNK // 16, hi, 0)
            return [
                pltpu.async_copy(tables_hbm.at[ibuf.at[buf, j]], gbuf.at[buf, j],
                                 gsems[buf])
                for j in range(16)
            ]

        def accum(l, buf, descs):
            for d in descs:
                d.wait()

            def ai(i, _):
                ds = pl.ds(i * 16, 16)
                wx = (wbuf[buf, 0, ds], wbuf[buf, 1, ds])
                wy = (wbuf[buf, 2, ds], wbuf[buf, 3, ds])
                wz = (wbuf[buf, 4, ds], wbuf[buf, 5, ds])
                wyz = (wy[0] * wz[0], wy[1] * wz[0], wy[0] * wz[1], wy[1] * wz[1])
                acc0 = jnp.zeros((16,), jnp.float32)
                acc1 = jnp.zeros((16,), jnp.float32)
                for m in range(8):
                    wm = wx[m & 1] * wyz[m >> 1]
                    acc0 = acc0 + wm * gbuf[buf, 2 * m, ds]
                    acc1 = acc1 + wm * gbuf[buf, 2 * m + 1, ds]
                facc[2 * l, ds] = acc0
                facc[2 * l + 1, ds] = acc1
                return 0

            lax.fori_loop(0, CHUNK // 16, ai, 0)

        def chunk_body(ci, carry):
            base = wid * PTS_PER_TILE + ci * CHUNK
            pltpu.sync_copy(coords_hbm.at[pl.ds(base, CHUNK)], cbuf)
            descs = hash_fire(0, 0)
            for l in range(NUM_LEVEL):
                nxt = hash_fire(l + 1, (l + 1) & 1) if l + 1 < NUM_LEVEL else None
                accum(l, l & 1, descs)
                descs = nxt
            odescs = [
                pltpu.async_copy(
                    facc.at[r], feats_hbm.at[pl.ds(r * N_PTS + base, CHUNK)], osem)
                for r in range(2 * NUM_LEVEL)
            ]
            for d in odescs:
                d.wait()
            return carry

        lax.fori_loop(0, NCHUNK, chunk_body, 0)

    return k(coords, tables_words)


def _mlp_body(feats_ref, vd_ref,
              Wi, bi, Wh0, bh0, Wh1, bh1, Wo, bo,
              Wc1p, Wc2, bic, Wch0, bch0, Wch1, bch1, Wco, bco,
              out_ref):
    x = feats_ref[...]
    f32 = jnp.float32
    dn = (((0,), (0,)), ((), ()))
    h = jnp.maximum(lax.dot_general(x, Wi[...], dn, preferred_element_type=f32) + bi[...], 0.0)
    h = jnp.maximum(jnp.dot(h, Wh0[...], preferred_element_type=f32) + bh0[...], 0.0)
    h = jnp.maximum(jnp.dot(h, Wh1[...], preferred_element_type=f32) + bh1[...], 0.0)
    dout = jnp.dot(h, Wo[...], preferred_element_type=f32) + bo[...]
    c = (jnp.dot(dout, Wc1p[...], preferred_element_type=f32)
         + jnp.dot(vd_ref[...], Wc2[...], preferred_element_type=f32) + bic[...])
    c = jnp.maximum(c, 0.0)
    c = jnp.maximum(jnp.dot(c, Wch0[...], preferred_element_type=f32) + bch0[...], 0.0)
    c = jnp.maximum(jnp.dot(c, Wch1[...], preferred_element_type=f32) + bch1[...], 0.0)
    rgb = jnp.dot(c, Wco[...], preferred_element_type=f32) + bco[...]
    out_ref[...] = jnp.concatenate([dout[:, 0:1], rgb], axis=1)


def _mlp_call(feats_t, view_dirs, *weights):
    B = 2048
    grid = (N_PTS // B,)
    wspecs = [pl.BlockSpec(w.shape, lambda i: (0, 0)) for w in weights]
    return pl.pallas_call(
        _mlp_body,
        grid=grid,
        in_specs=[
            pl.BlockSpec((2 * NUM_LEVEL, B), lambda i: (0, i)),
            pl.BlockSpec((B, 3), lambda i: (i, 0)),
            *wspecs,
        ],
        out_specs=pl.BlockSpec((B, 4), lambda i: (i, 0)),
        out_shape=jax.ShapeDtypeStruct((N_PTS, 4), jnp.float32),
    )(feats_t, view_dirs, *weights)


def kernel(coords, view_dirs, tables,
           W_in_d, b_in_d, W_h0_d, b_h0_d, W_h1_d, b_h1_d, W_out_d, b_out_d,
           W_in_c, b_in_c, W_h0_c, b_h0_c, W_h1_c, b_h1_c, W_out_c, b_out_c):
    tables_words = tables.reshape(NUM_LEVEL * T * FEAT_DIM)
    feats_t = _sc_embed(coords, tables_words).reshape(2 * NUM_LEVEL, N_PTS)
    Wc1p = jnp.concatenate(
        [jnp.zeros((1, HIDDEN), jnp.float32), W_in_c[: GEO_DIM - 1]], axis=0)
    Wc2 = W_in_c[GEO_DIM - 1:]
    r = lambda b: b.reshape(1, -1)
    return _mlp_call(
        feats_t, view_dirs,
        W_in_d, r(b_in_d), W_h0_d, r(b_h0_d), W_h1_d, r(b_h1_d), W_out_d, r(b_out_d),
        Wc1p, Wc2, r(b_in_c), W_h0_c, r(b_h0_c), W_h1_c, r(b_h1_c), W_out_c, r(b_out_c))


# speedup vs baseline: 4.3897x; 4.3897x over previous
import functools

import numpy as np
import jax
import jax.numpy as jnp
from jax import lax
from jax.experimental import pallas as pl
from jax.experimental.pallas import tpu as pltpu
from jax.experimental.pallas import tpu_sc as plsc

NUM_LEVEL = 16
T = 2 ** 19
FEAT_DIM = 2
N_PTS = 262144
GEO_DIM = 16
HIDDEN = 64

NC, NS = 2, 16
NW = NC * NS
PTS_PER_TILE = N_PTS // NW
CHUNK = 1024
NCHUNK = PTS_PER_TILE // CHUNK
TMASK = T - 1
HC1 = int(np.int32(np.uint32(2654435761)))
HC2 = int(np.int32(np.uint32(805459861)))
_RES = [float(r) for r in np.floor(16.0 * (128.0 ** (1.0 / 15.0)) ** np.arange(16))]


def _sc_embed(coords, tables_words):
    mesh = plsc.VectorSubcoreMesh(core_axis_name="c", subcore_axis_name="s")

    @functools.partial(
        pl.kernel,
        out_type=jax.ShapeDtypeStruct((2 * NUM_LEVEL * N_PTS,), jnp.float32),
        mesh=mesh,
        scratch_types=[
            pltpu.VMEM((CHUNK, 3), jnp.float32),
            pltpu.VMEM((2, 6, CHUNK), jnp.float32),
            pltpu.VMEM((2, 16, CHUNK), jnp.int32),
            pltpu.VMEM((2, 16, CHUNK), jnp.float32),
            pltpu.VMEM((2 * NUM_LEVEL, CHUNK), jnp.float32),
            pltpu.SemaphoreType.DMA,
            pltpu.SemaphoreType.DMA,
            pltpu.SemaphoreType.DMA,
        ],
        compiler_params=pltpu.CompilerParams(
            use_tc_tiling_on_sc=False, needs_layout_passes=False),
    )
    def k(coords_hbm, tables_hbm, feats_hbm, cbuf, wbuf, ibuf, gbuf, facc,
          gsem0, gsem1, osem):
        wid = lax.axis_index("s") * NC + lax.axis_index("c")
        gsems = (gsem0, gsem1)
        iota16 = lax.iota(jnp.int32, 16)
        cd0 = jnp.zeros((16,), jnp.int32)
        cd1 = jnp.full((16,), 1, jnp.int32)
        cd2 = jnp.full((16,), 2, jnp.int32)

        def hash_fire(l, buf):
            res = _RES[l]
            off2 = 2 * l * T

            def hi(i, _):
                ds = pl.ds(i * 16, 16)
                pidx = iota16 + i * 16
                sx = plsc.load_gather(cbuf, [pidx, cd0]) * res
                sy = plsc.load_gather(cbuf, [pidx, cd1]) * res
                sz = plsc.load_gather(cbuf, [pidx, cd2]) * res
                fx = sx.astype(jnp.int32)
                fy = sy.astype(jnp.int32)
                fz = sz.astype(jnp.int32)
                frx = sx - fx.astype(jnp.float32)
                fry = sy - fy.astype(jnp.float32)
                frz = sz - fz.astype(jnp.float32)
                wbuf[buf, 0, ds] = 1.0 - frx
                wbuf[buf, 1, ds] = frx
                wbuf[buf, 2, ds] = 1.0 - fry
                wbuf[buf, 3, ds] = fry
                wbuf[buf, 4, ds] = 1.0 - frz
                wbuf[buf, 5, ds] = frz
                hy0 = fy * HC1
                hy1 = hy0 + HC1
                hz0 = fz * HC2
                hz1 = hz0 + HC2
                hx1 = fx + 1
                a = (fx ^ hy0, hx1 ^ hy0, fx ^ hy1, hx1 ^ hy1)
                for m in range(8):
                    hxy = a[m & 3]
                    hz = hz0 if (m >> 2) & 1 == 0 else hz1
                    h = (hxy ^ hz) & TMASK
                    w0 = ((h << 1) - (h & 127)) + off2
                    ibuf[buf, 2 * m, ds] = w0
                    ibuf[buf, 2 * m + 1, ds] = w0 + 128
                return 0

            lax.fori_loop(0, CHUNK // 16, hi, 0)
            return [
                pltpu.async_copy(tables_hbm.at[ibuf.at[buf, j]], gbuf.at[buf, j],
                                 gsems[buf])
                for j in range(16)
            ]

        def accum(l, buf, descs):
            for d in descs:
                d.wait()

            def ai(i, _):
                ds = pl.ds(i * 16, 16)
                wx = (wbuf[buf, 0, ds], wbuf[buf, 1, ds])
                wy = (wbuf[buf, 2, ds], wbuf[buf, 3, ds])
                wz = (wbuf[buf, 4, ds], wbuf[buf, 5, ds])
                wyz = (wy[0] * wz[0], wy[1] * wz[0], wy[0] * wz[1], wy[1] * wz[1])
                acc0 = jnp.zeros((16,), jnp.float32)
                acc1 = jnp.zeros((16,), jnp.float32)
                for m in range(8):
                    wm = wx[m & 1] * wyz[m >> 1]
                    acc0 = acc0 + wm * gbuf[buf, 2 * m, ds]
                    acc1 = acc1 + wm * gbuf[buf, 2 * m + 1, ds]
                facc[2 * l, ds] = acc0
                facc[2 * l + 1, ds] = acc1
                return 0

            lax.fori_loop(0, CHUNK // 16, ai, 0)

        def chunk_body(ci, carry):
            base = wid * PTS_PER_TILE + ci * CHUNK
            pltpu.sync_copy(coords_hbm.at[pl.ds(base, CHUNK)], cbuf)
            descs = hash_fire(0, 0)
            for l in range(NUM_LEVEL):
                nxt = hash_fire(l + 1, (l + 1) & 1) if l + 1 < NUM_LEVEL else None
                accum(l, l & 1, descs)
                descs = nxt
            odescs = [
                pltpu.async_copy(
                    facc.at[r], feats_hbm.at[pl.ds(r * N_PTS + base, CHUNK)], osem)
                for r in range(2 * NUM_LEVEL)
            ]
            for d in odescs:
                d.wait()
            return carry

        lax.fori_loop(0, NCHUNK, chunk_body, 0)

    return k(coords, tables_words)


def _mlp_body(feats_ref, vd_ref,
              Wi, bi, Wh0, bh0, Wh1, bh1, Wo, bo,
              Wc1p, Wc2, bic, Wch0, bch0, Wch1, bch1, Wco, bco,
              out_ref):
    x = feats_ref[...]
    f32 = jnp.float32
    dn = (((0,), (0,)), ((), ()))
    h = jnp.maximum(lax.dot_general(x, Wi[...], dn, preferred_element_type=f32) + bi[...], 0.0)
    h = jnp.maximum(jnp.dot(h, Wh0[...], preferred_element_type=f32) + bh0[...], 0.0)
    h = jnp.maximum(jnp.dot(h, Wh1[...], preferred_element_type=f32) + bh1[...], 0.0)
    dout = jnp.dot(h, Wo[...], preferred_element_type=f32) + bo[...]
    c = (jnp.dot(dout, Wc1p[...], preferred_element_type=f32)
         + jnp.dot(vd_ref[...], Wc2[...], preferred_element_type=f32) + bic[...])
    c = jnp.maximum(c, 0.0)
    c = jnp.maximum(jnp.dot(c, Wch0[...], preferred_element_type=f32) + bch0[...], 0.0)
    c = jnp.maximum(jnp.dot(c, Wch1[...], preferred_element_type=f32) + bch1[...], 0.0)
    rgb = jnp.dot(c, Wco[...], preferred_element_type=f32) + bco[...]
    out_ref[...] = jnp.concatenate([dout[:, 0:1], rgb], axis=1)


def _mlp_call(feats_t, view_dirs, *weights):
    B = 2048
    grid = (N_PTS // B,)
    wspecs = [pl.BlockSpec(w.shape, lambda i: (0, 0)) for w in weights]
    return pl.pallas_call(
        _mlp_body,
        grid=grid,
        in_specs=[
            pl.BlockSpec((2 * NUM_LEVEL, B), lambda i: (0, i)),
            pl.BlockSpec((B, 3), lambda i: (i, 0)),
            *wspecs,
        ],
        out_specs=pl.BlockSpec((B, 4), lambda i: (i, 0)),
        out_shape=jax.ShapeDtypeStruct((N_PTS, 4), jnp.float32),
    )(feats_t, view_dirs, *weights)


def kernel(coords, view_dirs, tables,
           W_in_d, b_in_d, W_h0_d, b_h0_d, W_h1_d, b_h1_d, W_out_d, b_out_d,
           W_in_c, b_in_c, W_h0_c, b_h0_c, W_h1_c, b_h1_c, W_out_c, b_out_c):
    tables_words = tables.reshape(NUM_LEVEL, T // 128, 128, FEAT_DIM) \
                         .transpose(0, 1, 3, 2).reshape(-1)
    feats_t = _sc_embed(coords, tables_words).reshape(2 * NUM_LEVEL, N_PTS)
    Wc1p = jnp.concatenate(
        [jnp.zeros((1, HIDDEN), jnp.float32), W_in_c[: GEO_DIM - 1]], axis=0)
    Wc2 = W_in_c[GEO_DIM - 1:]
    r = lambda b: b.reshape(1, -1)
    return _mlp_call(
        feats_t, view_dirs,
        W_in_d, r(b_in_d), W_h0_d, r(b_h0_d), W_h1_d, r(b_h1_d), W_out_d, r(b_out_d),
        Wc1p, Wc2, r(b_in_c), W_h0_c, r(b_h0_c), W_h1_c, r(b_h1_c), W_out_c, r(b_out_c))


# speedup vs baseline: 7.3589x; 1.6764x over previous
import functools

import numpy as np
import jax
import jax.numpy as jnp
from jax import lax
from jax.experimental import pallas as pl
from jax.experimental.pallas import tpu as pltpu
from jax.experimental.pallas import tpu_sc as plsc

NUM_LEVEL = 16
T = 2 ** 19
FEAT_DIM = 2
N_PTS = 262144
GEO_DIM = 16
HIDDEN = 64

NC, NS = 2, 16
NW = NC * NS
PTS_PER_TILE = N_PTS // NW
CHUNK = 512
NCHUNK = PTS_PER_TILE // CHUNK
TMASK = T - 1
HC1 = int(np.int32(np.uint32(2654435761)))
HC2 = int(np.int32(np.uint32(805459861)))
_RES = [float(r) for r in np.floor(16.0 * (128.0 ** (1.0 / 15.0)) ** np.arange(16))]


def _sc_embed(coords, tables_words):
    mesh = plsc.VectorSubcoreMesh(core_axis_name="c", subcore_axis_name="s")

    WPL = 2 * T
    STAGE = WPL // NS

    @functools.partial(
        pl.kernel,
        out_type=jax.ShapeDtypeStruct((2 * NUM_LEVEL * N_PTS,), jnp.float32),
        mesh=mesh,
        scratch_types=[
            pltpu.VMEM((CHUNK, 3), jnp.float32),
            pltpu.VMEM((6, CHUNK), jnp.float32),
            pltpu.VMEM((16, CHUNK), jnp.int32),
            pltpu.VMEM((16, CHUNK), jnp.float32),
            pltpu.VMEM((2, CHUNK), jnp.float32),
            pltpu.VMEM_SHARED((WPL,), jnp.float32),
            pltpu.SemaphoreType.DMA,
            pltpu.SemaphoreType.DMA,
        ],
        compiler_params=pltpu.CompilerParams(
            use_tc_tiling_on_sc=False, needs_layout_passes=False),
    )
    def k(coords_hbm, tables_hbm, feats_hbm, cbuf, wbuf, ibuf, gbuf, facc,
          spmem, gsem, osem):
        sid = lax.axis_index("s")
        wid = sid * NC + lax.axis_index("c")
        iota16 = lax.iota(jnp.int32, 16)
        cd0 = jnp.zeros((16,), jnp.int32)
        cd1 = jnp.full((16,), 1, jnp.int32)
        cd2 = jnp.full((16,), 2, jnp.int32)

        def hash_fire(l, ci):
            res = _RES[l]
            pltpu.sync_copy(
                coords_hbm.at[pl.ds(wid * PTS_PER_TILE + ci * CHUNK, CHUNK)], cbuf)

            def hi(i, _):
                ds = pl.ds(i * 16, 16)
                pidx = iota16 + i * 16
                sx = plsc.load_gather(cbuf, [pidx, cd0]) * res
                sy = plsc.load_gather(cbuf, [pidx, cd1]) * res
                sz = plsc.load_gather(cbuf, [pidx, cd2]) * res
                fx = sx.astype(jnp.int32)
                fy = sy.astype(jnp.int32)
                fz = sz.astype(jnp.int32)
                frx = sx - fx.astype(jnp.float32)
                fry = sy - fy.astype(jnp.float32)
                frz = sz - fz.astype(jnp.float32)
                wbuf[0, ds] = 1.0 - frx
                wbuf[1, ds] = frx
                wbuf[2, ds] = 1.0 - fry
                wbuf[3, ds] = fry
                wbuf[4, ds] = 1.0 - frz
                wbuf[5, ds] = frz
                hy0 = fy * HC1
                hy1 = hy0 + HC1
                hz0 = fz * HC2
                hz1 = hz0 + HC2
                hx1 = fx + 1
                a = (fx ^ hy0, hx1 ^ hy0, fx ^ hy1, hx1 ^ hy1)
                for m in range(8):
                    hxy = a[m & 3]
                    hz = hz0 if (m >> 2) & 1 == 0 else hz1
                    h = (hxy ^ hz) & TMASK
                    w0 = (h << 1) - (h & 127)
                    ibuf[2 * m, ds] = w0
                    ibuf[2 * m + 1, ds] = w0 + 128
                return 0

            lax.fori_loop(0, CHUNK // 16, hi, 0)
            return [
                pltpu.async_copy(spmem.at[ibuf.at[j]], gbuf.at[j], gsem)
                for j in range(16)
            ]

        def accum(l, ci, descs):
            for d in descs:
                d.wait()

            def ai(i, _):
                ds = pl.ds(i * 16, 16)
                wx = (wbuf[0, ds], wbuf[1, ds])
                wy = (wbuf[2, ds], wbuf[3, ds])
                wz = (wbuf[4, ds], wbuf[5, ds])
                wyz = (wy[0] * wz[0], wy[1] * wz[0], wy[0] * wz[1], wy[1] * wz[1])
                acc0 = jnp.zeros((16,), jnp.float32)
                acc1 = jnp.zeros((16,), jnp.float32)
                for m in range(8):
                    wm = wx[m & 1] * wyz[m >> 1]
                    acc0 = acc0 + wm * gbuf[2 * m, ds]
                    acc1 = acc1 + wm * gbuf[2 * m + 1, ds]
                facc[0, ds] = acc0
                facc[1, ds] = acc1
                return 0

            lax.fori_loop(0, CHUNK // 16, ai, 0)
            base = wid * PTS_PER_TILE + ci * CHUNK
            d0 = pltpu.async_copy(
                facc.at[0], feats_hbm.at[pl.ds(2 * l * N_PTS + base, CHUNK)], osem)
            d1 = pltpu.async_copy(
                facc.at[1],
                feats_hbm.at[pl.ds((2 * l + 1) * N_PTS + base, CHUNK)], osem)
            d0.wait()
            d1.wait()

        for l in range(NUM_LEVEL):
            pltpu.sync_copy(
                tables_hbm.at[pl.ds(l * WPL + sid * STAGE, STAGE)],
                spmem.at[pl.ds(sid * STAGE, STAGE)])
            plsc.subcore_barrier()

            def level_chunks(ci, carry):
                descs = hash_fire(l, ci)
                accum(l, ci, descs)
                return carry

            lax.fori_loop(0, NCHUNK, level_chunks, 0)
            plsc.subcore_barrier()

    return k(coords, tables_words)


def _mlp_body(feats_ref, vd_ref,
              Wi, bi, Wh0, bh0, Wh1, bh1, Wo, bo,
              Wc1p, Wc2, bic, Wch0, bch0, Wch1, bch1, Wco, bco,
              out_ref):
    x = feats_ref[...]
    f32 = jnp.float32
    dn = (((0,), (0,)), ((), ()))
    h = jnp.maximum(lax.dot_general(x, Wi[...], dn, preferred_element_type=f32) + bi[...], 0.0)
    h = jnp.maximum(jnp.dot(h, Wh0[...], preferred_element_type=f32) + bh0[...], 0.0)
    h = jnp.maximum(jnp.dot(h, Wh1[...], preferred_element_type=f32) + bh1[...], 0.0)
    dout = jnp.dot(h, Wo[...], preferred_element_type=f32) + bo[...]
    c = (jnp.dot(dout, Wc1p[...], preferred_element_type=f32)
         + jnp.dot(vd_ref[...], Wc2[...], preferred_element_type=f32) + bic[...])
    c = jnp.maximum(c, 0.0)
    c = jnp.maximum(jnp.dot(c, Wch0[...], preferred_element_type=f32) + bch0[...], 0.0)
    c = jnp.maximum(jnp.dot(c, Wch1[...], preferred_element_type=f32) + bch1[...], 0.0)
    rgb = jnp.dot(c, Wco[...], preferred_element_type=f32) + bco[...]
    out_ref[...] = jnp.concatenate([dout[:, 0:1], rgb], axis=1)


def _mlp_call(feats_t, view_dirs, *weights):
    B = 2048
    grid = (N_PTS // B,)
    wspecs = [pl.BlockSpec(w.shape, lambda i: (0, 0)) for w in weights]
    return pl.pallas_call(
        _mlp_body,
        grid=grid,
        in_specs=[
            pl.BlockSpec((2 * NUM_LEVEL, B), lambda i: (0, i)),
            pl.BlockSpec((B, 3), lambda i: (i, 0)),
            *wspecs,
        ],
        out_specs=pl.BlockSpec((B, 4), lambda i: (i, 0)),
        out_shape=jax.ShapeDtypeStruct((N_PTS, 4), jnp.float32),
    )(feats_t, view_dirs, *weights)


def kernel(coords, view_dirs, tables,
           W_in_d, b_in_d, W_h0_d, b_h0_d, W_h1_d, b_h1_d, W_out_d, b_out_d,
           W_in_c, b_in_c, W_h0_c, b_h0_c, W_h1_c, b_h1_c, W_out_c, b_out_c):
    tables_words = tables.reshape(NUM_LEVEL, T // 128, 128, FEAT_DIM) \
                         .transpose(0, 1, 3, 2).reshape(-1)
    feats_t = _sc_embed(coords, tables_words).reshape(2 * NUM_LEVEL, N_PTS)
    Wc1p = jnp.concatenate(
        [jnp.zeros((1, HIDDEN), jnp.float32), W_in_c[: GEO_DIM - 1]], axis=0)
    Wc2 = W_in_c[GEO_DIM - 1:]
    r = lambda b: b.reshape(1, -1)
    return _mlp_call(
        feats_t, view_dirs,
        W_in_d, r(b_in_d), W_h0_d, r(b_h0_d), W_h1_d, r(b_h1_d), W_out_d, r(b_out_d),
        Wc1p, Wc2, r(b_in_c), W_h0_c, r(b_h0_c), W_h1_c, r(b_h1_c), W_out_c, r(b_out_c))


# speedup vs baseline: 10.0230x; 1.3620x over previous
import functools

import numpy as np
import jax
import jax.numpy as jnp
from jax import lax
from jax.experimental import pallas as pl
from jax.experimental.pallas import tpu as pltpu
from jax.experimental.pallas import tpu_sc as plsc

NUM_LEVEL = 16
T = 2 ** 19
FEAT_DIM = 2
N_PTS = 262144
GEO_DIM = 16
HIDDEN = 64

NC, NS = 2, 16
NW = NC * NS
PTS_PER_TILE = N_PTS // NW
CHUNK = 512
NCHUNK = PTS_PER_TILE // CHUNK
TMASK = T - 1
HC1 = int(np.int32(np.uint32(2654435761)))
HC2 = int(np.int32(np.uint32(805459861)))
_RES = [float(r) for r in np.floor(16.0 * (128.0 ** (1.0 / 15.0)) ** np.arange(16))]


def _sc_embed(coords, tables_words):
    mesh = plsc.VectorSubcoreMesh(core_axis_name="c", subcore_axis_name="s")

    WPL = 2 * T
    STAGE = WPL // NS
    NPAIR = NCHUNK // 2

    @functools.partial(
        pl.kernel,
        out_type=jax.ShapeDtypeStruct((2 * NUM_LEVEL * N_PTS,), jnp.float32),
        mesh=mesh,
        scratch_types=[
            pltpu.VMEM((CHUNK, 3), jnp.float32),
            pltpu.VMEM((2, 6, CHUNK), jnp.float32),
            pltpu.VMEM((2, 16, CHUNK), jnp.int32),
            pltpu.VMEM((2, 16, CHUNK), jnp.float32),
            pltpu.VMEM((2, CHUNK), jnp.float32),
            pltpu.VMEM((16,), jnp.float32),
            pltpu.VMEM_SHARED((WPL,), jnp.float32),
            pltpu.SemaphoreType.DMA,
            pltpu.SemaphoreType.DMA,
            pltpu.SemaphoreType.DMA,
        ],
        compiler_params=pltpu.CompilerParams(
            use_tc_tiling_on_sc=False, needs_layout_passes=False),
    )
    def k(coords_hbm, tables_hbm, res_hbm, feats_hbm, cbuf, wbuf, ibuf, gbuf,
          facc, resb, spmem, gsem0, gsem1, osem):
        sid = lax.axis_index("s")
        wid = sid * NC + lax.axis_index("c")
        iota16 = lax.iota(jnp.int32, 16)
        cd0 = jnp.zeros((16,), jnp.int32)
        cd1 = jnp.full((16,), 1, jnp.int32)
        cd2 = jnp.full((16,), 2, jnp.int32)
        gsems = (gsem0, gsem1)
        pltpu.sync_copy(res_hbm, resb)

        def hash_fire(res, ci, buf):
            pltpu.sync_copy(
                coords_hbm.at[pl.ds(wid * PTS_PER_TILE + ci * CHUNK, CHUNK)], cbuf)

            def hi(i, _):
                ds = pl.ds(i * 16, 16)
                pidx = iota16 + i * 16
                sx = plsc.load_gather(cbuf, [pidx, cd0]) * res
                sy = plsc.load_gather(cbuf, [pidx, cd1]) * res
                sz = plsc.load_gather(cbuf, [pidx, cd2]) * res
                fx = sx.astype(jnp.int32)
                fy = sy.astype(jnp.int32)
                fz = sz.astype(jnp.int32)
                frx = sx - fx.astype(jnp.float32)
                fry = sy - fy.astype(jnp.float32)
                frz = sz - fz.astype(jnp.float32)
                wbuf[buf, 0, ds] = 1.0 - frx
                wbuf[buf, 1, ds] = frx
                wbuf[buf, 2, ds] = 1.0 - fry
                wbuf[buf, 3, ds] = fry
                wbuf[buf, 4, ds] = 1.0 - frz
                wbuf[buf, 5, ds] = frz
                hy0 = fy * HC1
                hy1 = hy0 + HC1
                hz0 = fz * HC2
                hz1 = hz0 + HC2
                hx1 = fx + 1
                a = (fx ^ hy0, hx1 ^ hy0, fx ^ hy1, hx1 ^ hy1)
                for m in range(8):
                    hxy = a[m & 3]
                    hz = hz0 if (m >> 2) & 1 == 0 else hz1
                    h = (hxy ^ hz) & TMASK
                    w0 = (h << 1) - (h & 127)
                    ibuf[buf, 2 * m, ds] = w0
                    ibuf[buf, 2 * m + 1, ds] = w0 + 128
                return 0

            lax.fori_loop(0, CHUNK // 16, hi, 0)
            for j in range(16):
                pltpu.async_copy(spmem.at[ibuf.at[buf, j]], gbuf.at[buf, j],
                                 gsems[buf])

        def wait_gathers(buf):
            for j in range(16):
                pltpu.make_async_copy(spmem.at[ibuf.at[buf, j]], gbuf.at[buf, j],
                                      gsems[buf]).wait()

        def accum(lofs, ci, buf):
            def ai(i, _):
                ds = pl.ds(i * 16, 16)
                wx = (wbuf[buf, 0, ds], wbuf[buf, 1, ds])
                wy = (wbuf[buf, 2, ds], wbuf[buf, 3, ds])
                wz = (wbuf[buf, 4, ds], wbuf[buf, 5, ds])
                wyz = (wy[0] * wz[0], wy[1] * wz[0], wy[0] * wz[1], wy[1] * wz[1])
                acc0 = jnp.zeros((16,), jnp.float32)
                acc1 = jnp.zeros((16,), jnp.float32)
                for m in range(8):
                    wm = wx[m & 1] * wyz[m >> 1]
                    acc0 = acc0 + wm * gbuf[buf, 2 * m, ds]
                    acc1 = acc1 + wm * gbuf[buf, 2 * m + 1, ds]
                facc[0, ds] = acc0
                facc[1, ds] = acc1
                return 0

            lax.fori_loop(0, CHUNK // 16, ai, 0)
            base = wid * PTS_PER_TILE + ci * CHUNK
            d0 = pltpu.async_copy(
                facc.at[0], feats_hbm.at[pl.ds(lofs + base, CHUNK)], osem)
            d1 = pltpu.async_copy(
                facc.at[1], feats_hbm.at[pl.ds(lofs + N_PTS + base, CHUNK)], osem)
            d0.wait()
            d1.wait()

        def level_body(l, carry):
            res = plsc.load_gather(resb, [jnp.full((16,), 0, jnp.int32) + l])
            lofs = 2 * l * N_PTS
            pltpu.sync_copy(
                tables_hbm.at[pl.ds(l * WPL + sid * STAGE, STAGE)],
                spmem.at[pl.ds(sid * STAGE, STAGE)])
            plsc.subcore_barrier()
            hash_fire(res, 0, 0)

            def pair_body(j, carry2):
                hash_fire(res, 2 * j + 1, 1)
                wait_gathers(0)
                accum(lofs, 2 * j, 0)

                @pl.when(j < NPAIR - 1)
                def _():
                    hash_fire(res, 2 * j + 2, 0)

                wait_gathers(1)
                accum(lofs, 2 * j + 1, 1)
                return carry2

            lax.fori_loop(0, NPAIR, pair_body, 0)
            plsc.subcore_barrier()
            return carry

        lax.fori_loop(0, NUM_LEVEL, level_body, 0)

    return k(coords, tables_words, jnp.asarray(_RES, jnp.float32))


def _mlp_body(feats_ref, vd_ref,
              Wi, bi, Wh0, bh0, Wh1, bh1, Wo, bo,
              Wc1p, Wc2, bic, Wch0, bch0, Wch1, bch1, Wco, bco,
              out_ref):
    x = feats_ref[...]
    f32 = jnp.float32
    dn = (((0,), (0,)), ((), ()))
    h = jnp.maximum(lax.dot_general(x, Wi[...], dn, preferred_element_type=f32) + bi[...], 0.0)
    h = jnp.maximum(jnp.dot(h, Wh0[...], preferred_element_type=f32) + bh0[...], 0.0)
    h = jnp.maximum(jnp.dot(h, Wh1[...], preferred_element_type=f32) + bh1[...], 0.0)
    dout = jnp.dot(h, Wo[...], preferred_element_type=f32) + bo[...]
    c = (jnp.dot(dout, Wc1p[...], preferred_element_type=f32)
         + jnp.dot(vd_ref[...], Wc2[...], preferred_element_type=f32) + bic[...])
    c = jnp.maximum(c, 0.0)
    c = jnp.maximum(jnp.dot(c, Wch0[...], preferred_element_type=f32) + bch0[...], 0.0)
    c = jnp.maximum(jnp.dot(c, Wch1[...], preferred_element_type=f32) + bch1[...], 0.0)
    rgb = jnp.dot(c, Wco[...], preferred_element_type=f32) + bco[...]
    out_ref[...] = jnp.concatenate([dout[:, 0:1], rgb], axis=1)


def _mlp_call(feats_t, view_dirs, *weights):
    B = 2048
    grid = (N_PTS // B,)
    wspecs = [pl.BlockSpec(w.shape, lambda i: (0, 0)) for w in weights]
    return pl.pallas_call(
        _mlp_body,
        grid=grid,
        in_specs=[
            pl.BlockSpec((2 * NUM_LEVEL, B), lambda i: (0, i)),
            pl.BlockSpec((B, 3), lambda i: (i, 0)),
            *wspecs,
        ],
        out_specs=pl.BlockSpec((B, 4), lambda i: (i, 0)),
        out_shape=jax.ShapeDtypeStruct((N_PTS, 4), jnp.float32),
    )(feats_t, view_dirs, *weights)


def kernel(coords, view_dirs, tables,
           W_in_d, b_in_d, W_h0_d, b_h0_d, W_h1_d, b_h1_d, W_out_d, b_out_d,
           W_in_c, b_in_c, W_h0_c, b_h0_c, W_h1_c, b_h1_c, W_out_c, b_out_c):
    tables_words = tables.reshape(NUM_LEVEL, T // 128, 128, FEAT_DIM) \
                         .transpose(0, 1, 3, 2).reshape(-1)
    feats_t = _sc_embed(coords, tables_words).reshape(2 * NUM_LEVEL, N_PTS)
    Wc1p = jnp.concatenate(
        [jnp.zeros((1, HIDDEN), jnp.float32), W_in_c[: GEO_DIM - 1]], axis=0)
    Wc2 = W_in_c[GEO_DIM - 1:]
    r = lambda b: b.reshape(1, -1)
    return _mlp_call(
        feats_t, view_dirs,
        W_in_d, r(b_in_d), W_h0_d, r(b_h0_d), W_h1_d, r(b_h1_d), W_out_d, r(b_out_d),
        Wc1p, Wc2, r(b_in_c), W_h0_c, r(b_h0_c), W_h1_c, r(b_h1_c), W_out_c, r(b_out_c))


# speedup vs baseline: 10.6055x; 1.0581x over previous
import functools

import numpy as np
import jax
import jax.numpy as jnp
from jax import lax
from jax.experimental import pallas as pl
from jax.experimental.pallas import tpu as pltpu
from jax.experimental.pallas import tpu_sc as plsc

NUM_LEVEL = 16
T = 2 ** 19
FEAT_DIM = 2
N_PTS = 262144
GEO_DIM = 16
HIDDEN = 64

NC, NS = 2, 16
NW = NC * NS
PTS_PER_TILE = N_PTS // NW
CHUNK = 512
NCHUNK = PTS_PER_TILE // CHUNK
TMASK = T - 1
HC1 = int(np.int32(np.uint32(2654435761)))
HC2 = int(np.int32(np.uint32(805459861)))
_RES = [float(r) for r in np.floor(16.0 * (128.0 ** (1.0 / 15.0)) ** np.arange(16))]


def _sc_embed(coords, view_dirs, tables_words):
    mesh = plsc.VectorSubcoreMesh(core_axis_name="c", subcore_axis_name="s")

    WPL = 2 * T
    STAGE = WPL // NS
    NPAIR = NCHUNK // 2

    @functools.partial(
        pl.kernel,
        out_type=jax.ShapeDtypeStruct(((2 * NUM_LEVEL + 3) * N_PTS,), jnp.float32),
        mesh=mesh,
        scratch_types=[
            pltpu.VMEM((3, CHUNK), jnp.float32),
            pltpu.VMEM((2, 6, CHUNK), jnp.float32),
            pltpu.VMEM((2, 16, CHUNK), jnp.int32),
            pltpu.VMEM((2, 16, CHUNK), jnp.float32),
            pltpu.VMEM((2, CHUNK), jnp.float32),
            pltpu.VMEM((16,), jnp.float32),
            pltpu.VMEM_SHARED((WPL,), jnp.float32),
            pltpu.SemaphoreType.DMA,
            pltpu.SemaphoreType.DMA,
            pltpu.SemaphoreType.DMA,
        ],
        compiler_params=pltpu.CompilerParams(
            use_tc_tiling_on_sc=False, needs_layout_passes=False),
    )
    def k(xs_hbm, ys_hbm, zs_hbm, vx_hbm, vy_hbm, vz_hbm, tables_hbm, res_hbm,
          feats_hbm, cbuf, wbuf, ibuf, gbuf, facc, resb, spmem,
          gsem0, gsem1, osem):
        sid = lax.axis_index("s")
        wid = sid * NC + lax.axis_index("c")
        iota16 = lax.iota(jnp.int32, 16)
        gsems = (gsem0, gsem1)
        pltpu.sync_copy(res_hbm, resb)
        tofs = wid * PTS_PER_TILE
        for r, v in enumerate((vx_hbm, vy_hbm, vz_hbm)):
            pltpu.sync_copy(
                v.at[pl.ds(tofs, PTS_PER_TILE)],
                feats_hbm.at[pl.ds((2 * NUM_LEVEL + r) * N_PTS + tofs,
                                   PTS_PER_TILE)])

        def hash_fire(res, ci, buf):
            cofs = wid * PTS_PER_TILE + ci * CHUNK
            pltpu.sync_copy(xs_hbm.at[pl.ds(cofs, CHUNK)], cbuf.at[0])
            pltpu.sync_copy(ys_hbm.at[pl.ds(cofs, CHUNK)], cbuf.at[1])
            pltpu.sync_copy(zs_hbm.at[pl.ds(cofs, CHUNK)], cbuf.at[2])

            def hi(i, _):
                ds = pl.ds(i * 16, 16)
                sx = cbuf[0, ds] * res
                sy = cbuf[1, ds] * res
                sz = cbuf[2, ds] * res
                fx = sx.astype(jnp.int32)
                fy = sy.astype(jnp.int32)
                fz = sz.astype(jnp.int32)
                frx = sx - fx.astype(jnp.float32)
                fry = sy - fy.astype(jnp.float32)
                frz = sz - fz.astype(jnp.float32)
                wbuf[buf, 0, ds] = 1.0 - frx
                wbuf[buf, 1, ds] = frx
                wbuf[buf, 2, ds] = 1.0 - fry
                wbuf[buf, 3, ds] = fry
                wbuf[buf, 4, ds] = 1.0 - frz
                wbuf[buf, 5, ds] = frz
                hy0 = fy * HC1
                hy1 = hy0 + HC1
                hz0 = fz * HC2
                hz1 = hz0 + HC2
                hx1 = fx + 1
                a = (fx ^ hy0, hx1 ^ hy0, fx ^ hy1, hx1 ^ hy1)
                for m in range(8):
                    hxy = a[m & 3]
                    hz = hz0 if (m >> 2) & 1 == 0 else hz1
                    h = (hxy ^ hz) & TMASK
                    w0 = (h << 1) - (h & 127)
                    ibuf[buf, 2 * m, ds] = w0
                    ibuf[buf, 2 * m + 1, ds] = w0 + 128
                return 0

            lax.fori_loop(0, CHUNK // 16, hi, 0)
            for j in range(16):
                pltpu.async_copy(spmem.at[ibuf.at[buf, j]], gbuf.at[buf, j],
                                 gsems[buf])

        def wait_gathers(buf):
            for j in range(16):
                pltpu.make_async_copy(spmem.at[ibuf.at[buf, j]], gbuf.at[buf, j],
                                      gsems[buf]).wait()

        def accum(lofs, ci, buf):
            def ai(i, _):
                ds = pl.ds(i * 16, 16)
                wx = (wbuf[buf, 0, ds], wbuf[buf, 1, ds])
                wy = (wbuf[buf, 2, ds], wbuf[buf, 3, ds])
                wz = (wbuf[buf, 4, ds], wbuf[buf, 5, ds])
                wyz = (wy[0] * wz[0], wy[1] * wz[0], wy[0] * wz[1], wy[1] * wz[1])
                acc0 = jnp.zeros((16,), jnp.float32)
                acc1 = jnp.zeros((16,), jnp.float32)
                for m in range(8):
                    wm = wx[m & 1] * wyz[m >> 1]
                    acc0 = acc0 + wm * gbuf[buf, 2 * m, ds]
                    acc1 = acc1 + wm * gbuf[buf, 2 * m + 1, ds]
                facc[0, ds] = acc0
                facc[1, ds] = acc1
                return 0

            lax.fori_loop(0, CHUNK // 16, ai, 0)
            base = wid * PTS_PER_TILE + ci * CHUNK
            d0 = pltpu.async_copy(
                facc.at[0], feats_hbm.at[pl.ds(lofs + base, CHUNK)], osem)
            d1 = pltpu.async_copy(
                facc.at[1], feats_hbm.at[pl.ds(lofs + N_PTS + base, CHUNK)], osem)
            d0.wait()
            d1.wait()

        def level_body(l, carry):
            res = plsc.load_gather(resb, [jnp.full((16,), 0, jnp.int32) + l])
            lofs = 2 * l * N_PTS
            pltpu.sync_copy(
                tables_hbm.at[pl.ds(l * WPL + sid * STAGE, STAGE)],
                spmem.at[pl.ds(sid * STAGE, STAGE)])
            plsc.subcore_barrier()
            hash_fire(res, 0, 0)

            def pair_body(j, carry2):
                hash_fire(res, 2 * j + 1, 1)
                wait_gathers(0)
                accum(lofs, 2 * j, 0)

                @pl.when(j < NPAIR - 1)
                def _():
                    hash_fire(res, 2 * j + 2, 0)

                wait_gathers(1)
                accum(lofs, 2 * j + 1, 1)
                return carry2

            lax.fori_loop(0, NPAIR, pair_body, 0)
            plsc.subcore_barrier()
            return carry

        lax.fori_loop(0, NUM_LEVEL, level_body, 0)

    return k(coords[:, 0], coords[:, 1], coords[:, 2],
             view_dirs[:, 0], view_dirs[:, 1], view_dirs[:, 2],
             tables_words, jnp.asarray(_RES, jnp.float32))


def _mlp_body(feats_ref,
              Wi, bi, Wh0, bh0, Wh1, bh1, Wo, bo,
              Wc1p, Wc2, bic, Wch0, bch0, Wch1, bch1, Wco, bco,
              out_ref):
    x = feats_ref[...]
    f32 = jnp.float32
    dn = (((0,), (0,)), ((), ()))
    h = jnp.maximum(
        lax.dot_general(x[:2 * NUM_LEVEL], Wi[...], dn,
                        preferred_element_type=f32) + bi[...], 0.0)
    h = jnp.maximum(jnp.dot(h, Wh0[...], preferred_element_type=f32) + bh0[...], 0.0)
    h = jnp.maximum(jnp.dot(h, Wh1[...], preferred_element_type=f32) + bh1[...], 0.0)
    dout = jnp.dot(h, Wo[...], preferred_element_type=f32) + bo[...]
    c = (jnp.dot(dout, Wc1p[...], preferred_element_type=f32)
         + lax.dot_general(x[2 * NUM_LEVEL:], Wc2[...], dn,
                           preferred_element_type=f32) + bic[...])
    c = jnp.maximum(c, 0.0)
    c = jnp.maximum(jnp.dot(c, Wch0[...], preferred_element_type=f32) + bch0[...], 0.0)
    c = jnp.maximum(jnp.dot(c, Wch1[...], preferred_element_type=f32) + bch1[...], 0.0)
    rgb = jnp.dot(c, Wco[...], preferred_element_type=f32) + bco[...]
    out_ref[...] = jnp.concatenate([dout[:, 0:1], rgb], axis=1)


def _mlp_call(feats_t, *weights):
    B = 2048
    grid = (N_PTS // B,)
    wspecs = [pl.BlockSpec(w.shape, lambda i: (0, 0)) for w in weights]
    return pl.pallas_call(
        _mlp_body,
        grid=grid,
        in_specs=[
            pl.BlockSpec((2 * NUM_LEVEL + 3, B), lambda i: (0, i)),
            *wspecs,
        ],
        out_specs=pl.BlockSpec((B, 4), lambda i: (i, 0)),
        out_shape=jax.ShapeDtypeStruct((N_PTS, 4), jnp.float32),
    )(feats_t, *weights)


def kernel(coords, view_dirs, tables,
           W_in_d, b_in_d, W_h0_d, b_h0_d, W_h1_d, b_h1_d, W_out_d, b_out_d,
           W_in_c, b_in_c, W_h0_c, b_h0_c, W_h1_c, b_h1_c, W_out_c, b_out_c):
    tables_words = tables.reshape(NUM_LEVEL, T // 128, 128, FEAT_DIM) \
                         .transpose(0, 1, 3, 2).reshape(-1)
    feats_t = _sc_embed(coords, view_dirs, tables_words) \
        .reshape(2 * NUM_LEVEL + 3, N_PTS)
    Wc1p = jnp.concatenate(
        [jnp.zeros((1, HIDDEN), jnp.float32), W_in_c[: GEO_DIM - 1]], axis=0)
    Wc2 = W_in_c[GEO_DIM - 1:]
    r = lambda b: b.reshape(1, -1)
    return _mlp_call(
        feats_t,
        W_in_d, r(b_in_d), W_h0_d, r(b_h0_d), W_h1_d, r(b_h1_d), W_out_d, r(b_out_d),
        Wc1p, Wc2, r(b_in_c), W_h0_c, r(b_h0_c), W_h1_c, r(b_h1_c), W_out_c, r(b_out_c))


# speedup vs baseline: 10.9010x; 1.0279x over previous
import functools

import numpy as np
import jax
import jax.numpy as jnp
from jax import lax
from jax.experimental import pallas as pl
from jax.experimental.pallas import tpu as pltpu
from jax.experimental.pallas import tpu_sc as plsc

NUM_LEVEL = 16
T = 2 ** 19
FEAT_DIM = 2
N_PTS = 262144
GEO_DIM = 16
HIDDEN = 64

NC, NS = 2, 16
NW = NC * NS
PTS_PER_TILE = N_PTS // NW
CHUNK = 512
NCHUNK = PTS_PER_TILE // CHUNK
TMASK = T - 1
HC1 = int(np.int32(np.uint32(2654435761)))
HC2 = int(np.int32(np.uint32(805459861)))
_RES = [float(r) for r in np.floor(16.0 * (128.0 ** (1.0 / 15.0)) ** np.arange(16))]


def _sc_embed(coords, view_dirs, tables_words, npts):
    mesh = plsc.VectorSubcoreMesh(core_axis_name="c", subcore_axis_name="s")

    WPL = 2 * T
    STAGE = WPL // NS
    ppt = npts // NW
    nchunk = ppt // CHUNK
    NPAIR = nchunk // 2

    @functools.partial(
        pl.kernel,
        out_type=jax.ShapeDtypeStruct(((2 * NUM_LEVEL + 3) * npts,), jnp.float32),
        mesh=mesh,
        scratch_types=[
            pltpu.VMEM((3, CHUNK), jnp.float32),
            pltpu.VMEM((2, 6, CHUNK), jnp.float32),
            pltpu.VMEM((2, 16, CHUNK), jnp.int32),
            pltpu.VMEM((2, 16, CHUNK), jnp.float32),
            pltpu.VMEM((2, CHUNK), jnp.float32),
            pltpu.VMEM((16,), jnp.float32),
            pltpu.VMEM_SHARED((WPL,), jnp.float32),
            pltpu.SemaphoreType.DMA,
            pltpu.SemaphoreType.DMA,
            pltpu.SemaphoreType.DMA,
        ],
        compiler_params=pltpu.CompilerParams(
            use_tc_tiling_on_sc=False, needs_layout_passes=False),
    )
    def k(xs_hbm, ys_hbm, zs_hbm, vx_hbm, vy_hbm, vz_hbm, tables_hbm, res_hbm,
          feats_hbm, cbuf, wbuf, ibuf, gbuf, facc, resb, spmem,
          gsem0, gsem1, osem):
        sid = lax.axis_index("s")
        wid = sid * NC + lax.axis_index("c")
        iota16 = lax.iota(jnp.int32, 16)
        gsems = (gsem0, gsem1)
        pltpu.sync_copy(res_hbm, resb)
        tofs = wid * ppt
        for r, v in enumerate((vx_hbm, vy_hbm, vz_hbm)):
            pltpu.sync_copy(
                v.at[pl.ds(tofs, ppt)],
                feats_hbm.at[pl.ds((2 * NUM_LEVEL + r) * npts + tofs, ppt)])

        def hash_fire(res, ci, buf):
            cofs = wid * ppt + ci * CHUNK
            pltpu.sync_copy(xs_hbm.at[pl.ds(cofs, CHUNK)], cbuf.at[0])
            pltpu.sync_copy(ys_hbm.at[pl.ds(cofs, CHUNK)], cbuf.at[1])
            pltpu.sync_copy(zs_hbm.at[pl.ds(cofs, CHUNK)], cbuf.at[2])

            def hi(i, _):
                ds = pl.ds(i * 16, 16)
                sx = cbuf[0, ds] * res
                sy = cbuf[1, ds] * res
                sz = cbuf[2, ds] * res
                fx = sx.astype(jnp.int32)
                fy = sy.astype(jnp.int32)
                fz = sz.astype(jnp.int32)
                frx = sx - fx.astype(jnp.float32)
                fry = sy - fy.astype(jnp.float32)
                frz = sz - fz.astype(jnp.float32)
                wbuf[buf, 0, ds] = 1.0 - frx
                wbuf[buf, 1, ds] = frx
                wbuf[buf, 2, ds] = 1.0 - fry
                wbuf[buf, 3, ds] = fry
                wbuf[buf, 4, ds] = 1.0 - frz
                wbuf[buf, 5, ds] = frz
                hy0 = fy * HC1
                hy1 = hy0 + HC1
                hz0 = fz * HC2
                hz1 = hz0 + HC2
                hx1 = fx + 1
                a = (fx ^ hy0, hx1 ^ hy0, fx ^ hy1, hx1 ^ hy1)
                for m in range(8):
                    hxy = a[m & 3]
                    hz = hz0 if (m >> 2) & 1 == 0 else hz1
                    h = (hxy ^ hz) & TMASK
                    w0 = (h << 1) - (h & 127)
                    ibuf[buf, 2 * m, ds] = w0
                    ibuf[buf, 2 * m + 1, ds] = w0 + 128
                return 0

            lax.fori_loop(0, CHUNK // 16, hi, 0)
            for j in range(16):
                pltpu.async_copy(spmem.at[ibuf.at[buf, j]], gbuf.at[buf, j],
                                 gsems[buf])

        def wait_gathers(buf):
            for j in range(16):
                pltpu.make_async_copy(spmem.at[ibuf.at[buf, j]], gbuf.at[buf, j],
                                      gsems[buf]).wait()

        def accum(lofs, ci, buf):
            def ai(i, _):
                ds = pl.ds(i * 16, 16)
                wx = (wbuf[buf, 0, ds], wbuf[buf, 1, ds])
                wy = (wbuf[buf, 2, ds], wbuf[buf, 3, ds])
                wz = (wbuf[buf, 4, ds], wbuf[buf, 5, ds])
                wyz = (wy[0] * wz[0], wy[1] * wz[0], wy[0] * wz[1], wy[1] * wz[1])
                acc0 = jnp.zeros((16,), jnp.float32)
                acc1 = jnp.zeros((16,), jnp.float32)
                for m in range(8):
                    wm = wx[m & 1] * wyz[m >> 1]
                    acc0 = acc0 + wm * gbuf[buf, 2 * m, ds]
                    acc1 = acc1 + wm * gbuf[buf, 2 * m + 1, ds]
                facc[0, ds] = acc0
                facc[1, ds] = acc1
                return 0

            lax.fori_loop(0, CHUNK // 16, ai, 0)
            base = wid * ppt + ci * CHUNK
            d0 = pltpu.async_copy(
                facc.at[0], feats_hbm.at[pl.ds(lofs + base, CHUNK)], osem)
            d1 = pltpu.async_copy(
                facc.at[1], feats_hbm.at[pl.ds(lofs + npts + base, CHUNK)], osem)
            d0.wait()
            d1.wait()

        def level_body(l, carry):
            res = plsc.load_gather(resb, [jnp.full((16,), 0, jnp.int32) + l])
            lofs = 2 * l * npts
            pltpu.sync_copy(
                tables_hbm.at[pl.ds(l * WPL + sid * STAGE, STAGE)],
                spmem.at[pl.ds(sid * STAGE, STAGE)])
            plsc.subcore_barrier()
            hash_fire(res, 0, 0)

            def pair_body(j, carry2):
                hash_fire(res, 2 * j + 1, 1)
                wait_gathers(0)
                accum(lofs, 2 * j, 0)

                @pl.when(j < NPAIR - 1)
                def _():
                    hash_fire(res, 2 * j + 2, 0)

                wait_gathers(1)
                accum(lofs, 2 * j + 1, 1)
                return carry2

            lax.fori_loop(0, NPAIR, pair_body, 0)
            plsc.subcore_barrier()
            return carry

        lax.fori_loop(0, NUM_LEVEL, level_body, 0)

    return k(coords[:, 0], coords[:, 1], coords[:, 2],
             view_dirs[:, 0], view_dirs[:, 1], view_dirs[:, 2],
             tables_words, jnp.asarray(_RES, jnp.float32))


def _mlp_body(feats_ref,
              Wi, bi, Wh0, bh0, Wh1, bh1, Wo, bo,
              Wc1p, Wc2, bic, Wch0, bch0, Wch1, bch1, Wco, bco,
              out_ref):
    x = feats_ref[...]
    f32 = jnp.float32
    dn = (((0,), (0,)), ((), ()))
    h = jnp.maximum(
        lax.dot_general(x[:2 * NUM_LEVEL], Wi[...], dn,
                        preferred_element_type=f32) + bi[...], 0.0)
    h = jnp.maximum(jnp.dot(h, Wh0[...], preferred_element_type=f32) + bh0[...], 0.0)
    h = jnp.maximum(jnp.dot(h, Wh1[...], preferred_element_type=f32) + bh1[...], 0.0)
    dout = jnp.dot(h, Wo[...], preferred_element_type=f32) + bo[...]
    c = (jnp.dot(dout, Wc1p[...], preferred_element_type=f32)
         + lax.dot_general(x[2 * NUM_LEVEL:], Wc2[...], dn,
                           preferred_element_type=f32) + bic[...])
    c = jnp.maximum(c, 0.0)
    c = jnp.maximum(jnp.dot(c, Wch0[...], preferred_element_type=f32) + bch0[...], 0.0)
    c = jnp.maximum(jnp.dot(c, Wch1[...], preferred_element_type=f32) + bch1[...], 0.0)
    rgb = jnp.dot(c, Wco[...], preferred_element_type=f32) + bco[...]
    out_ref[...] = jnp.concatenate([dout[:, 0:1], rgb], axis=1)


def _mlp_call(feats_t, npts, *weights):
    B = 2048
    grid = (npts // B,)
    wspecs = [pl.BlockSpec(w.shape, lambda i: (0, 0)) for w in weights]
    return pl.pallas_call(
        _mlp_body,
        grid=grid,
        in_specs=[
            pl.BlockSpec((2 * NUM_LEVEL + 3, B), lambda i: (0, i)),
            *wspecs,
        ],
        out_specs=pl.BlockSpec((B, 4), lambda i: (i, 0)),
        out_shape=jax.ShapeDtypeStruct((npts, 4), jnp.float32),
    )(feats_t, *weights)


def kernel(coords, view_dirs, tables,
           W_in_d, b_in_d, W_h0_d, b_h0_d, W_h1_d, b_h1_d, W_out_d, b_out_d,
           W_in_c, b_in_c, W_h0_c, b_h0_c, W_h1_c, b_h1_c, W_out_c, b_out_c):
    tables_words = tables.reshape(NUM_LEVEL, T // 128, 128, FEAT_DIM) \
                         .transpose(0, 1, 3, 2).reshape(-1)
    Wc1p = jnp.concatenate(
        [jnp.zeros((1, HIDDEN), jnp.float32), W_in_c[: GEO_DIM - 1]], axis=0)
    Wc2 = W_in_c[GEO_DIM - 1:]
    r = lambda b: b.reshape(1, -1)
    ws = (W_in_d, r(b_in_d), W_h0_d, r(b_h0_d), W_h1_d, r(b_h1_d), W_out_d,
          r(b_out_d), Wc1p, Wc2, r(b_in_c), W_h0_c, r(b_h0_c), W_h1_c,
          r(b_h1_c), W_out_c, r(b_out_c))
    H = N_PTS // 2
    outs = []
    for h in range(2):
        sl = slice(h * H, (h + 1) * H)
        feats_t = _sc_embed(coords[sl], view_dirs[sl], tables_words, H) \
            .reshape(2 * NUM_LEVEL + 3, H)
        outs.append(_mlp_call(feats_t, H, *ws))
    return jnp.concatenate(outs, axis=0)
